# baseline (speedup 1.0000x reference)
import jax
import jax.numpy as jnp
from jax import lax
from jax.experimental import pallas as pl
from jax.experimental.pallas import tpu as pltpu
from jax.experimental.pallas import tpu_sc as plsc

_B, _L, _H = 1024, 200, 768
_T = _B * _L
_NC, _NS = 2, 16
_NW = _NC * _NS
_TPW = _T // _NW
_C = 16
_R = _C * 6
_SUP = 25
_SUPR = _SUP * _R
_NSUP = _TPW // (_C * _SUP)
_EPS = 1e-12


def _lane_sum(v):
    idx = lax.iota(jnp.int32, 16)
    dnums = lax.GatherDimensionNumbers(
        offset_dims=(), collapsed_slice_dims=(0,), start_index_map=(0,))
    for d in (8, 4, 2, 1):
        perm = jnp.bitwise_xor(idx, jnp.int32(d))
        v = v + lax.gather(v, perm[:, None], dnums, slice_sizes=(1,),
                           mode=lax.GatherScatterMode.PROMISE_IN_BOUNDS)
    return v


def _rsqrt_vec(v):
    bits = lax.bitcast_convert_type(v, jnp.int32)
    y = lax.bitcast_convert_type(
        jnp.int32(0x5F3759DF) - lax.shift_right_logical(bits, 1), jnp.float32)
    for _ in range(3):
        y = y * (1.5 - 0.5 * v * y * y)
    return y


def _zero_acc(acc):
    def zj(j, c):
        def zi(i, c2):
            acc[0, i, pl.ds(j * 16, 16)] = jnp.zeros((16,), jnp.float32)
            return c2
        return lax.fori_loop(0, _R, zi, c)
    lax.fori_loop(0, 8, zj, 0)


def _body(x0, x1, x2, x3, x4, x5, w6, h6, e6, m6, u6, p6, gamma, beta,
          out_hbm, i0, i1, i2, i3, i4, i5, acc, obuf, gb_v, sem):
    wid = lax.axis_index("s") * _NC + lax.axis_index("c")
    base6 = wid * _TPW * 6

    pltpu.sync_copy(gamma, gb_v.at[0])
    pltpu.sync_copy(beta, gb_v.at[1])
    _zero_acc(acc)

    def super_body(s, carry):
        soff = base6 + s * _SUPR
        pltpu.sync_copy(x0.at[pl.ds(soff, _SUPR)], i0)
        pltpu.sync_copy(x1.at[pl.ds(soff, _SUPR)], i1)
        pltpu.sync_copy(x2.at[pl.ds(soff, _SUPR)], i2)
        pltpu.sync_copy(x3.at[pl.ds(soff, _SUPR)], i3)
        pltpu.sync_copy(x4.at[pl.ds(soff, _SUPR)], i4)
        pltpu.sync_copy(x5.at[pl.ds(soff, _SUPR)], i5)

        def chunk_body(c, carry2):
            off = c * _R
            d = pltpu.async_copy(w6.at[i0.at[pl.ds(off, _R)]], acc.at[0],
                                 sem, add=True)
            pltpu.async_copy(h6.at[i1.at[pl.ds(off, _R)]], acc.at[0],
                             sem, add=True)
            pltpu.async_copy(e6.at[i2.at[pl.ds(off, _R)]], acc.at[0],
                             sem, add=True)
            pltpu.async_copy(m6.at[i3.at[pl.ds(off, _R)]], acc.at[0],
                             sem, add=True)
            pltpu.async_copy(u6.at[i4.at[pl.ds(off, _R)]], acc.at[0],
                             sem, add=True)
            pltpu.async_copy(p6.at[i5.at[pl.ds(off, _R)]], acc.at[0],
                             sem, add=True)
            for _ in range(6):
                d.wait()

            def ln_one(i, c3):
                def pass1(k, pc):
                    s_, q_ = pc
                    v = acc[0, i * 6 + (k // 8), pl.ds((k % 8) * 16, 16)]
                    return s_ + v, q_ + v * v

                z = jnp.zeros((16,), jnp.float32)
                s_, q_ = lax.fori_loop(0, 48, pass1, (z, z))
                mu = _lane_sum(s_) * (1.0 / _H)
                var = _lane_sum(q_) * (1.0 / _H) - mu * mu
                rs = _rsqrt_vec(var + _EPS)

                def pass2(k, c4):
                    r = i * 6 + (k // 8)
                    cs = (k % 8) * 16
                    v = acc[0, r, pl.ds(cs, 16)]
                    gk = gb_v[0, k // 8, pl.ds(cs, 16)]
                    bk = gb_v[1, k // 8, pl.ds(cs, 16)]
                    obuf[0, r, pl.ds(cs, 16)] = (v - mu) * rs * gk + bk
                    acc[0, r, pl.ds(cs, 16)] = jnp.zeros((16,), jnp.float32)
                    return c4

                return lax.fori_loop(0, 48, pass2, c3)

            lax.fori_loop(0, _C, ln_one, 0)
            g = s * _SUP + c
            pltpu.sync_copy(obuf.at[0],
                            out_hbm.at[pl.ds((wid * _TPW + g * _C) * 6, _R)])
            return carry2

        lax.fori_loop(0, _SUP, chunk_body, 0)
        return carry

    lax.fori_loop(0, _NSUP, super_body, 0)


@jax.jit
def _launch(idxs, w6, h6, e6, m6, u6, p6, gamma, beta):
    mesh = plsc.VectorSubcoreMesh(core_axis_name="c", subcore_axis_name="s")
    run = pl.kernel(
        _body,
        out_type=jax.ShapeDtypeStruct((_T * 6, 128), jnp.float32),
        mesh=mesh,
        scratch_types=(
            [pltpu.VMEM((_SUPR,), jnp.int32) for _ in range(6)]
            + [pltpu.VMEM((1, _R, 128), jnp.float32),
               pltpu.VMEM((1, _R, 128), jnp.float32),
               pltpu.VMEM((2, 6, 128), jnp.float32),
               pltpu.SemaphoreType.DMA]),
    )
    return run(*idxs, w6, h6, e6, m6, u6, p6, gamma, beta)


def kernel(input_ids, hashtag_ids, emoji_ids, mention_ids, url_flags, task_id,
           word_emb, pos_emb, hashtag_emb, emoji_emb, mention_emb, url_emb,
           task_emb, ln_gamma, ln_beta):
    zero = jnp.zeros((1, _H), jnp.float32)
    htab = jnp.concatenate([zero, hashtag_emb[1:]], axis=0)
    etab = jnp.concatenate([zero, emoji_emb[1:]], axis=0)
    mtab = jnp.concatenate([zero, mention_emb[1:]], axis=0)
    utab = jnp.concatenate([zero, url_emb[1:]], axis=0)
    ptab = pos_emb[:_L] + task_emb[task_id][None, :]

    pos_ids = jnp.broadcast_to(jnp.arange(_L, dtype=jnp.int32), (_B, _L))
    ids = jnp.stack([
        input_ids.reshape(-1).astype(jnp.int32),
        hashtag_ids.reshape(-1).astype(jnp.int32),
        emoji_ids.reshape(-1).astype(jnp.int32),
        mention_ids.reshape(-1).astype(jnp.int32),
        url_flags.reshape(-1).astype(jnp.int32),
        pos_ids.reshape(-1),
    ])
    sub = jnp.arange(6, dtype=jnp.int32)
    idx_stack = (ids[:, :, None] * 6 + sub[None, None, :]).reshape(6, _T * 6)
    idxs = tuple(idx_stack[j] for j in range(6))

    out = _launch(idxs, word_emb.reshape(-1, 128), htab.reshape(-1, 128),
                  etab.reshape(-1, 128), mtab.reshape(-1, 128),
                  utab.reshape(-1, 128), ptab.reshape(-1, 128),
                  ln_gamma.reshape(6, 128), ln_beta.reshape(6, 128))
    return out.reshape(_B, _L, _H)


# speedup vs baseline: 1.3055x; 1.3055x over previous
import jax
import jax.numpy as jnp
from jax import lax
from jax.experimental import pallas as pl
from jax.experimental.pallas import tpu as pltpu
from jax.experimental.pallas import tpu_sc as plsc

_B, _L, _H = 1024, 200, 768
_T = _B * _L
_NC, _NS = 2, 16
_NW = _NC * _NS
_TPW = _T // _NW
_C = 16
_R = _C * 6
_SUP = 25
_SUPR = _SUP * _R
_NCH = _TPW // _C
_NSUP = _NCH // _SUP
_EPS = 1e-12


def _lane_sum(v):
    idx = lax.iota(jnp.int32, 16)
    dnums = lax.GatherDimensionNumbers(
        offset_dims=(), collapsed_slice_dims=(0,), start_index_map=(0,))
    for d in (8, 4, 2, 1):
        perm = jnp.bitwise_xor(idx, jnp.int32(d))
        v = v + lax.gather(v, perm[:, None], dnums, slice_sizes=(1,),
                           mode=lax.GatherScatterMode.PROMISE_IN_BOUNDS)
    return v


def _rsqrt_vec(v):
    bits = lax.bitcast_convert_type(v, jnp.int32)
    y = lax.bitcast_convert_type(
        jnp.int32(0x5F3759DF) - lax.shift_right_logical(bits, 1), jnp.float32)
    for _ in range(3):
        y = y * (1.5 - 0.5 * v * y * y)
    return y


def _body(x0, x1, x2, x3, x4, x5, w6, h6, e6, m6, u6, p6, gamma, beta,
          out_hbm, ib, acc, obuf, gb_v, gsem, osem, isem):
    wid = lax.axis_index("s") * _NC + lax.axis_index("c")
    base6 = wid * _TPW * 6
    obase = wid * _TPW * 6
    xs = (x0, x1, x2, x3, x4, x5)

    pltpu.sync_copy(gamma, gb_v.at[0])
    pltpu.sync_copy(beta, gb_v.at[1])

    def zj(j, c):
        def zi(i, c2):
            acc[c2, i, pl.ds(j * 16, 16)] = jnp.zeros((16,), jnp.float32)
            return c2
        return lax.fori_loop(0, _R, zi, c)
    lax.fori_loop(0, 8, zj, 0)
    lax.fori_loop(0, 8, zj, 1)

    _SLOT = 6 * _SUPR

    def stage_idx_sync(s):
        for j, x in enumerate(xs):
            pltpu.sync_copy(x.at[pl.ds(base6 + s * _SUPR, _SUPR)],
                            ib.at[pl.ds((s % 2) * _SLOT + j * _SUPR, _SUPR)])

    def stage_idx_async(s):
        for j, x in enumerate(xs):
            pltpu.async_copy(x.at[pl.ds(base6 + s * _SUPR, _SUPR)],
                             ib.at[pl.ds((s % 2) * _SLOT + j * _SUPR, _SUPR)],
                             isem)

    def wait_idx():
        for j in range(6):
            pltpu.make_async_copy(x0.at[pl.ds(0, _SUPR)],
                                  ib.at[pl.ds(j * _SUPR, _SUPR)], isem).wait()

    def issue_gathers(t, buf):
        slot = (t // _SUP) % 2
        off = (t % _SUP) * _R
        tabs = (w6, h6, e6, m6, u6, p6)
        for j, tab in enumerate(tabs):
            ix = ib.at[pl.ds(slot * _SLOT + j * _SUPR + off, _R)]
            pltpu.async_copy(tab.at[ix], acc.at[buf], gsem.at[buf], add=True)

    def wait_gathers(buf):
        for _ in range(6):
            pltpu.make_async_copy(w6.at[pl.ds(0, _R)], acc.at[buf],
                                  gsem.at[buf]).wait()

    def wait_out(buf):
        pltpu.make_async_copy(obuf.at[buf], out_hbm.at[pl.ds(0, _R)],
                              osem.at[buf]).wait()

    stage_idx_sync(0)
    stage_idx_async(1)
    issue_gathers(0, 0)
    issue_gathers(1, 1)

    def chunk_body(g, carry):
        buf = g % 2
        t = g + 2
        st = t // _SUP
        ct = t % _SUP

        wait_gathers(buf)

        @pl.when(jnp.logical_and(ct == 1, st + 1 < _NSUP))
        def _():
            stage_idx_async(st + 1)

        def ln_one(i, c3):
            def pass1(k, pc):
                s_, q_ = pc
                v = acc[buf, i * 6 + (k // 8), pl.ds((k % 8) * 16, 16)]
                return s_ + v, q_ + v * v

            z = jnp.zeros((16,), jnp.float32)
            s_, q_ = lax.fori_loop(0, 48, pass1, (z, z))
            mu = _lane_sum(s_) * (1.0 / _H)
            var = _lane_sum(q_) * (1.0 / _H) - mu * mu
            rs = _rsqrt_vec(var + _EPS)

            def pass2(k, c4):
                r = i * 6 + (k // 8)
                cs = (k % 8) * 16
                v = acc[buf, r, pl.ds(cs, 16)]
                gk = gb_v[0, k // 8, pl.ds(cs, 16)]
                bk = gb_v[1, k // 8, pl.ds(cs, 16)]
                obuf[buf, r, pl.ds(cs, 16)] = (v - mu) * rs * gk + bk
                acc[buf, r, pl.ds(cs, 16)] = jnp.zeros((16,), jnp.float32)
                return c4

            return lax.fori_loop(0, 48, pass2, c3)

        lax.fori_loop(0, _C, ln_one, 0)

        @pl.when(g >= 2)
        def _():
            wait_out(buf)
        pltpu.async_copy(obuf.at[buf],
                         out_hbm.at[pl.ds(obase + g * _R, _R)],
                         osem.at[buf])

        @pl.when(t < _NCH)
        def _():
            @pl.when(ct == 0)
            def _():
                wait_idx()
            issue_gathers(t, buf)

        return carry

    lax.fori_loop(0, _NCH, chunk_body, 0)
    wait_out(0)
    wait_out(1)


@jax.jit
def _launch(idxs, w6, h6, e6, m6, u6, p6, gamma, beta):
    mesh = plsc.VectorSubcoreMesh(core_axis_name="c", subcore_axis_name="s")
    run = pl.kernel(
        _body,
        out_type=jax.ShapeDtypeStruct((_T * 6, 128), jnp.float32),
        mesh=mesh,
        scratch_types=(
            [pltpu.VMEM((2 * 6 * _SUPR,), jnp.int32),
             pltpu.VMEM((2, _R, 128), jnp.float32),
             pltpu.VMEM((2, _R, 128), jnp.float32),
             pltpu.VMEM((2, 6, 128), jnp.float32),
             pltpu.SemaphoreType.DMA((2,)),
             pltpu.SemaphoreType.DMA((2,)),
             pltpu.SemaphoreType.DMA]),
    )
    return run(*idxs, w6, h6, e6, m6, u6, p6, gamma, beta)


def kernel(input_ids, hashtag_ids, emoji_ids, mention_ids, url_flags, task_id,
           word_emb, pos_emb, hashtag_emb, emoji_emb, mention_emb, url_emb,
           task_emb, ln_gamma, ln_beta):
    zero = jnp.zeros((1, _H), jnp.float32)
    htab = jnp.concatenate([zero, hashtag_emb[1:]], axis=0)
    etab = jnp.concatenate([zero, emoji_emb[1:]], axis=0)
    mtab = jnp.concatenate([zero, mention_emb[1:]], axis=0)
    utab = jnp.concatenate([zero, url_emb[1:]], axis=0)
    ptab = pos_emb[:_L] + task_emb[task_id][None, :]

    pos_ids = jnp.broadcast_to(jnp.arange(_L, dtype=jnp.int32), (_B, _L))
    ids = jnp.stack([
        input_ids.reshape(-1).astype(jnp.int32),
        hashtag_ids.reshape(-1).astype(jnp.int32),
        emoji_ids.reshape(-1).astype(jnp.int32),
        mention_ids.reshape(-1).astype(jnp.int32),
        url_flags.reshape(-1).astype(jnp.int32),
        pos_ids.reshape(-1),
    ])
    sub = jnp.arange(6, dtype=jnp.int32)
    idx_stack = (ids[:, :, None] * 6 + sub[None, None, :]).reshape(6, _T * 6)
    idxs = tuple(idx_stack[j] for j in range(6))

    out = _launch(idxs, word_emb.reshape(-1, 128), htab.reshape(-1, 128),
                  etab.reshape(-1, 128), mtab.reshape(-1, 128),
                  utab.reshape(-1, 128), ptab.reshape(-1, 128),
                  ln_gamma.reshape(6, 128), ln_beta.reshape(6, 128))
    return out.reshape(_B, _L, _H)
